# baseline (speedup 1.0000x reference)
import jax
import jax.numpy as jnp
from jax import lax
from jax.experimental import pallas as pl
from jax.experimental.pallas import tpu as pltpu
from jax.experimental.pallas import tpu_sc as plsc

ZMAX = 87
NREF = 7
G_A = 3.0
G_C = 2.0
K2 = 4.0 / 3.0
K4 = 4.10451
K5 = 19.08857
K6 = 254.5553148552
KN = 7.5
WF = 6.0
BOHR = 0.5291772105638411
HARTREE = 27.211386024367243
C2B = 1.0 / BOHR
C2EV = 0.5 * HARTREE
EGA = 20.085536923187668

NA = 50000
NE = 800000
NW = 32
ATOMS_W = 1568
NA_PAD = NW * ATOMS_W
ZSL = NA_PAD // 16
CH = 512
RPC = CH // 128
EDGES_W = 25600
E_PAD = NW * EDGES_W
EROWS = E_PAD // 128
NC = EDGES_W // CH
NPAIR = ZMAX * ZMAX
N49 = ZMAX * 49
C6W = 56
RMAX = 4096


def _erf(x):
    ax = jnp.abs(x)
    t = 1.0 / (1.0 + 0.3275911 * ax)
    poly = t * (0.254829592 + t * (-0.284496736 + t * (1.421413741
               + t * (-1.453152027 + t * 1.061405429))))
    e = 1.0 - poly * jnp.exp(-x * x)
    return jnp.where(x >= 0, e, -e)


def _zero_f32(ref, n):
    def z16(i, carry):
        ref[pl.ds(i * 16, 16)] = jnp.zeros((16,), jnp.float32)
        return carry
    lax.fori_loop(0, n // 16, z16, 0)


def _lo_vec(los_v, w):
    return plsc.load_gather(los_v, [jnp.full((16,), w, jnp.int32)])


def _merge_acc(acc_v, ib_v, sh, lo, sem):
    ar16 = lax.iota(jnp.int32, 16)
    for b in range(RMAX // 128):
        for k in range(8):
            ib_v[b, pl.ds(k * 16, 16)] = lo + (b * 128 + k * 16) + ar16
    cps = [pltpu.async_copy(acc_v.at[pl.ds(b * 128, 128)],
                            sh.at[ib_v.at[b]], sem, add=True)
           for b in range(RMAX // 128)]
    for c in cps:
        c.wait()


def _readback(sl_v, sh, out, sid, cid):
    pltpu.sync_copy(sh.at[pl.ds(sid * ZSL, ZSL)], sl_v)
    pltpu.sync_copy(sl_v, out.at[pl.ds(cid * NA_PAD + sid * ZSL, ZSL)])


def _phase_a_body(zp, iip2, jjp2, rp2, rcot, dent, los, outc, outp,
                  z_v, rco_v, den_v, los_v,
                  iiA, jjA, rA, pbA, iiB, jjB, rB, pbB,
                  acc_v, ib_v, sl_v, cov_sh, semA, semB, semM):
    cid = lax.axis_index("c")
    sid = lax.axis_index("s")
    w = cid * 16 + sid
    _zero_f32(sl_v, ZSL)
    pltpu.sync_copy(sl_v, cov_sh.at[pl.ds(sid * ZSL, ZSL)])
    _zero_f32(acc_v, RMAX)
    pltpu.sync_copy(zp, z_v)
    pltpu.sync_copy(rcot, rco_v)
    pltpu.sync_copy(dent, den_v)
    pltpu.sync_copy(los, los_v)
    lo = _lo_vec(los_v, w)
    plsc.subcore_barrier()

    def rows0(c):
        return w * (EDGES_W // 128) + jnp.minimum(c, NC - 1) * RPC

    def load(c, ii2, jj2, r2, sem, fire):
        r0 = rows0(c)
        args = [(iip2, ii2), (jjp2, jj2), (rp2, r2)]
        for src, dst in args:
            if fire:
                pltpu.async_copy(src.at[pl.ds(r0, RPC), :], dst, sem)
            else:
                pltpu.make_async_copy(src.at[pl.ds(r0, RPC), :], dst, sem).wait()

    def compute(c, ii2, jj2, r2, pb):
        def vreg(v, carry):
            j = v // 8
            o = (v % 8) * 16
            ii = ii2[j, pl.ds(o, 16)]
            jj = jj2[j, pl.ds(o, 16)]
            zi = plsc.load_gather(z_v, [ii])
            zj = plsc.load_gather(z_v, [jj])
            p = zi * ZMAX + zj
            pb[j, pl.ds(o, 16)] = p
            rcoi = plsc.load_gather(rco_v, [p])
            den = plsc.load_gather(den_v, [p])
            r = r2[j, pl.ds(o, 16)] * C2B
            tmp = den * 0.5 * (1.0 + _erf(KN * (1.0 - r * rcoi)))
            plsc.addupdate_scatter(acc_v, [ii - lo], tmp)
            return carry
        lax.fori_loop(0, CH // 16, vreg, 0)

    def pwrite(c, pb, sem, fire):
        dst = outp.at[pl.ds(rows0(c), RPC), :]
        if fire:
            pltpu.async_copy(pb, dst, sem)
        else:
            pltpu.make_async_copy(pb, dst, sem).wait()

    load(0, iiA, jjA, rA, semA, True)

    def pair(t, carry):
        c0 = 2 * t
        load(c0, iiA, jjA, rA, semA, False)
        load(c0 + 1, iiB, jjB, rB, semB, True)
        compute(c0, iiA, jjA, rA, pbA)
        pwrite(c0, pbA, semM, True)
        load(c0 + 1, iiB, jjB, rB, semB, False)
        load(c0 + 2, iiA, jjA, rA, semA, True)
        compute(c0 + 1, iiB, jjB, rB, pbB)
        pwrite(c0 + 1, pbB, semM, True)
        pwrite(c0, pbA, semM, False)
        pwrite(c0 + 1, pbB, semM, False)
        return carry

    lax.fori_loop(0, NC // 2, pair, 0)
    load(NC - 1, iiA, jjA, rA, semA, False)
    _merge_acc(acc_v, ib_v, cov_sh, lo, semM)
    plsc.subcore_barrier()
    _readback(sl_v, cov_sh, outc, sid, cid)


def _phase_b_body(cov2, zp, qap, wtt, cnt, fixgt, refqst, zefft, gamt,
                  kvec, out, covA, covB, zc, qac, wt_v, cn_v, fixg_v,
                  refqs_v, zeff_v, gam_v, kv_v, zbuf):
    w = lax.axis_index("c") * 16 + lax.axis_index("s")
    a0 = w * ATOMS_W
    pltpu.sync_copy(cov2.at[pl.ds(a0, ATOMS_W)], covA)
    pltpu.sync_copy(cov2.at[pl.ds(NA_PAD + a0, ATOMS_W)], covB)
    pltpu.sync_copy(zp.at[pl.ds(a0, ATOMS_W)], zc)
    pltpu.sync_copy(qap.at[pl.ds(a0, ATOMS_W)], qac)
    pltpu.sync_copy(wtt, wt_v)
    pltpu.sync_copy(cnt, cn_v)
    pltpu.sync_copy(fixgt, fixg_v)
    pltpu.sync_copy(refqst, refqs_v)
    pltpu.sync_copy(zefft, zeff_v)
    pltpu.sync_copy(gamt, gam_v)
    pltpu.sync_copy(kvec, kv_v)
    ar16 = lax.iota(jnp.int32, 16)

    def atoms(t, carry):
        o = t * 16
        cov = covA[pl.ds(o, 16)] + covB[pl.ds(o, 16)]
        zv = zc[pl.ds(o, 16)]
        b49 = zv * 49
        b7 = zv * NREF
        iz = plsc.load_gather(zeff_v, [zv])
        qmod = iz + qac[pl.ds(o, 16)]
        ok = qmod > 1e-8
        qm = jnp.where(ok, qmod, 1.0)
        g = plsc.load_gather(gam_v, [zv])
        gws = []
        norm = jnp.zeros((16,), jnp.float32)
        for i in range(NREF):
            acc = jnp.zeros((16,), jnp.float32)
            for j in range(NREF):
                idx = b49 + (NREF * i + j)
                wv = plsc.load_gather(wt_v, [idx])
                cnv = plsc.load_gather(cn_v, [idx])
                d = cov - cnv
                acc = acc + jnp.exp(-wv * d * d)
            gws.append(acc)
            norm = norm + acc
        nok = norm > 1e-8
        nrm = jnp.where(nok, norm, 1.0)
        kv = kv_v[pl.ds(0, 16)]
        zs = []
        for i in range(NREF):
            qref = iz + plsc.load_gather(refqs_v, [b7 + i])
            zt = jnp.where(ok, jnp.exp(G_A * (1.0 - jnp.exp(g * (1.0 - qref / qm)))), EGA)
            gwi = jnp.where(nok, gws[i] / nrm, plsc.load_gather(fixg_v, [b7 + i]))
            zs.append(zt * gwi * kv)
        zs.append(jnp.zeros((16,), jnp.float32))
        for m in range(4):
            zp_bf = plsc.pack(zs[2 * m], zs[2 * m + 1],
                              format=plsc.PackFormat.INTERLEAVED)
            word = plsc.bitcast(zp_bf, jnp.int32)
            plsc.store_scatter(zbuf, [o + ar16, jnp.full((16,), m, jnp.int32)], word)
        return carry

    lax.fori_loop(0, ATOMS_W // 16, atoms, 0)
    pltpu.sync_copy(zbuf, out.at[pl.ds(a0, ATOMS_W), :])


def _phase_c_body(iip2, jjp2, rp2, pp2, zeta, c6t, p6t, p8t, w8t, los, out,
                  p6_v, p8_v, w8_v, los_v,
                  iiA, jjA, ppA, rA, iiB, jjB, ppB, rB,
                  ziA, zjA, c6A, ziB, zjB, c6B,
                  acc_v, ib_v, sl_v, ed_sh,
                  semLA, semLB, semGA, semGB, semM):
    cid = lax.axis_index("c")
    sid = lax.axis_index("s")
    w = cid * 16 + sid
    _zero_f32(sl_v, ZSL)
    pltpu.sync_copy(sl_v, ed_sh.at[pl.ds(sid * ZSL, ZSL)])
    _zero_f32(acc_v, RMAX)
    pltpu.sync_copy(p6t, p6_v)
    pltpu.sync_copy(p8t, p8_v)
    pltpu.sync_copy(w8t, w8_v)
    pltpu.sync_copy(los, los_v)
    lo = _lo_vec(los_v, w)
    plsc.subcore_barrier()
    ar16 = lax.iota(jnp.int32, 16)

    def rows0(c):
        return w * (EDGES_W // 128) + jnp.minimum(c, NC - 1) * RPC

    def load(c, ii2, jj2, pp2v, r2, sem, fire):
        r0 = rows0(c)
        for src, dst in [(iip2, ii2), (jjp2, jj2), (pp2, pp2v), (rp2, r2)]:
            if fire:
                pltpu.async_copy(src.at[pl.ds(r0, RPC), :], dst, sem)
            else:
                pltpu.make_async_copy(src.at[pl.ds(r0, RPC), :], dst, sem).wait()

    def gath(ii2, jj2, pp2v, zi_v, zj_v, c6_v, sem, fire):
        for j in range(RPC):
            trips = [(zeta, ii2, zi_v, 8), (zeta, jj2, zj_v, 8),
                     (c6t, pp2v, c6_v, 32)]
            for tab, idx2, dst, _d in trips:
                src = tab.at[idx2.at[j]]
                dsl = dst.at[pl.ds(j * 128, 128), :]
                if fire:
                    pltpu.async_copy(src, dsl, sem)
                else:
                    pltpu.make_async_copy(src, dsl, sem).wait()

    def compute(ii2, pp2v, r2, zi_v, zj_v, c6_v):
        def one(v):
            j = v // 8
            o = (v % 8) * 16
            rows = v * 16 + ar16
            ii = ii2[j, pl.ds(o, 16)]
            p = pp2v[j, pl.ds(o, 16)]
            r = r2[j, pl.ds(o, 16)] * C2B
            r2_ = r * r
            r4 = r2_ * r2_
            r6 = r4 * r2_
            r8 = r4 * r4
            p6 = plsc.load_gather(p6_v, [p])
            p8 = plsc.load_gather(p8_v, [p])
            w8 = plsc.load_gather(w8_v, [p])
            oor6 = 1.0 / (r6 + p6)
            oor8 = 1.0 / (r8 + p8)
            zjp = []
            zia = []
            for m in range(4):
                wj = plsc.load_gather(zj_v, [rows, jnp.full((16,), m, jnp.int32)])
                zjp.append(plsc.bitcast(wj, jnp.bfloat16))
                wi = plsc.load_gather(zi_v, [rows, jnp.full((16,), m, jnp.int32)])
                u0, u1 = plsc.unpack(plsc.bitcast(wi, jnp.bfloat16),
                                     format=plsc.PackFormat.INTERLEAVED)
                zia.append(u0.astype(jnp.float32))
                zia.append(u1.astype(jnp.float32))
            acc = jnp.zeros((32,), jnp.bfloat16)
            for a in range(NREF):
                ziap = plsc.pack(zia[a], zia[a],
                                 format=plsc.PackFormat.INTERLEAVED)
                for m in range(4):
                    crw = plsc.load_gather(
                        c6_v, [rows, jnp.full((16,), a * 4 + m, jnp.int32)])
                    crb = plsc.bitcast(crw, jnp.bfloat16)
                    acc = acc + (crb * zjp[m]) * ziap
            u0, u1 = plsc.unpack(acc, format=plsc.PackFormat.INTERLEAVED)
            c6 = u0.astype(jnp.float32) + u1.astype(jnp.float32)
            pw = -c6 * (oor6 + w8 * oor8)
            plsc.addupdate_scatter(acc_v, [ii - lo], pw)

        def vreg2(u, carry):
            one(2 * u)
            one(2 * u + 1)
            return carry
        lax.fori_loop(0, CH // 32, vreg2, 0)

    LA = (iiA, jjA, ppA, rA)
    LB = (iiB, jjB, ppB, rB)
    GA = (ziA, zjA, c6A)
    GB = (ziB, zjB, c6B)

    load(0, *LA, semLA, True)
    load(0, *LA, semLA, False)
    gath(iiA, jjA, ppA, *GA, semGA, True)
    load(1, *LB, semLB, True)

    def pair(t, carry):
        c0 = 2 * t
        load(c0 + 1, *LB, semLB, False)
        gath(iiB, jjB, ppB, *GB, semGB, True)
        gath(iiA, jjA, ppA, *GA, semGA, False)
        compute(iiA, ppA, rA, *GA)
        load(c0 + 2, *LA, semLA, True)
        load(c0 + 2, *LA, semLA, False)
        gath(iiA, jjA, ppA, *GA, semGA, True)
        gath(iiB, jjB, ppB, *GB, semGB, False)
        compute(iiB, ppB, rB, *GB)
        load(c0 + 3, *LB, semLB, True)
        return carry

    lax.fori_loop(0, NC // 2, pair, 0)
    load(NC - 1, *LB, semLB, False)
    gath(iiA, jjA, ppA, *GA, semGA, False)
    _merge_acc(acc_v, ib_v, ed_sh, lo, semM)
    plsc.subcore_barrier()
    _readback(sl_v, ed_sh, out, sid, cid)


def _build_phases():
    mesh = plsc.VectorSubcoreMesh(core_axis_name="c", subcore_axis_name="s")
    f32 = jnp.float32
    i32 = jnp.int32
    cparams = pltpu.CompilerParams(needs_layout_passes=False,
                                   use_tc_tiling_on_sc=False)
    phase_a = pl.kernel(
        _phase_a_body,
        out_type=(jax.ShapeDtypeStruct((2 * NA_PAD,), f32),
                  jax.ShapeDtypeStruct((EROWS, 128), i32)),
        compiler_params=cparams,
        mesh=mesh,
        scratch_types=[
            pltpu.VMEM((NA_PAD,), i32),
            pltpu.VMEM((NPAIR,), f32),
            pltpu.VMEM((NPAIR,), f32),
            pltpu.VMEM((NW,), i32),
            pltpu.VMEM((RPC, 128), i32),
            pltpu.VMEM((RPC, 128), i32),
            pltpu.VMEM((RPC, 128), f32),
            pltpu.VMEM((RPC, 128), i32),
            pltpu.VMEM((RPC, 128), i32),
            pltpu.VMEM((RPC, 128), i32),
            pltpu.VMEM((RPC, 128), f32),
            pltpu.VMEM((RPC, 128), i32),
            pltpu.VMEM((RMAX,), f32),
            pltpu.VMEM((RMAX // 128, 128), i32),
            pltpu.VMEM((ZSL,), f32),
            pltpu.VMEM_SHARED((NA_PAD,), f32),
            pltpu.SemaphoreType.DMA,
            pltpu.SemaphoreType.DMA,
            pltpu.SemaphoreType.DMA,
        ],
    )
    phase_b = pl.kernel(
        _phase_b_body,
        out_type=jax.ShapeDtypeStruct((NA_PAD, 8), i32),
        compiler_params=cparams,
        mesh=mesh,
        scratch_types=[
            pltpu.VMEM((ATOMS_W,), f32),
            pltpu.VMEM((ATOMS_W,), f32),
            pltpu.VMEM((ATOMS_W,), i32),
            pltpu.VMEM((ATOMS_W,), f32),
            pltpu.VMEM((N49,), f32),
            pltpu.VMEM((N49,), f32),
            pltpu.VMEM((ZMAX * NREF,), f32),
            pltpu.VMEM((ZMAX * NREF,), f32),
            pltpu.VMEM((ZMAX,), f32),
            pltpu.VMEM((ZMAX,), f32),
            pltpu.VMEM((16,), f32),
            pltpu.VMEM((ATOMS_W, 8), i32),
        ],
    )
    phase_c = pl.kernel(
        _phase_c_body,
        out_type=jax.ShapeDtypeStruct((2 * NA_PAD,), f32),
        compiler_params=cparams,
        mesh=mesh,
        scratch_types=[
            pltpu.VMEM((NPAIR,), f32),
            pltpu.VMEM((NPAIR,), f32),
            pltpu.VMEM((NPAIR,), f32),
            pltpu.VMEM((NW,), i32),
            pltpu.VMEM((RPC, 128), i32),
            pltpu.VMEM((RPC, 128), i32),
            pltpu.VMEM((RPC, 128), i32),
            pltpu.VMEM((RPC, 128), f32),
            pltpu.VMEM((RPC, 128), i32),
            pltpu.VMEM((RPC, 128), i32),
            pltpu.VMEM((RPC, 128), i32),
            pltpu.VMEM((RPC, 128), f32),
            pltpu.VMEM((CH, 8), i32),
            pltpu.VMEM((CH, 8), i32),
            pltpu.VMEM((CH, 32), i32),
            pltpu.VMEM((CH, 8), i32),
            pltpu.VMEM((CH, 8), i32),
            pltpu.VMEM((CH, 32), i32),
            pltpu.VMEM((RMAX,), f32),
            pltpu.VMEM((RMAX // 128, 128), i32),
            pltpu.VMEM((ZSL,), f32),
            pltpu.VMEM_SHARED((NA_PAD,), f32),
            pltpu.SemaphoreType.DMA,
            pltpu.SemaphoreType.DMA,
            pltpu.SemaphoreType.DMA,
            pltpu.SemaphoreType.DMA,
            pltpu.SemaphoreType.DMA,
        ],
    )
    return phase_a, phase_b, phase_c


def kernel(Z, idx_i, idx_j, r_ij, qa, s6_raw, s8_raw, a1_raw, a2_raw,
           scaleq_raw, refc6, rcov, en, ncount_mask, ncount_weight, cn,
           fixgweights, refq, zeff, gam, sqrt_r4r2):
    f32 = jnp.float32
    i32 = jnp.int32
    s6 = jax.nn.softplus(s6_raw)
    s8 = jax.nn.softplus(s8_raw)
    a1 = jax.nn.softplus(a1_raw)
    a2 = jax.nn.softplus(a2_raw)
    spq = jax.nn.softplus(scaleq_raw)

    rco_t = (1.0 / (K2 * (rcov[:, None] + rcov[None, :]))).reshape(-1)
    den_t = (K4 * jnp.exp(-(jnp.abs(en[:, None] - en[None, :]) + K5) ** 2 / K6)).reshape(-1)
    r4_t = (jnp.float32(3.0 ** 0.5) * sqrt_r4r2[:, None] * sqrt_r4r2[None, :]).reshape(-1)
    r0_t = a1 * r4_t + a2
    p6_t = r0_t ** 6
    p8_t = r0_t ** 8
    w8_t = s8 * r4_t * r4_t / s6
    kvec = jnp.full((16,), jnp.sqrt(s6 * C2EV), f32)

    wt_t = (WF * ncount_weight).reshape(-1)
    cn_t = jnp.where(ncount_mask.reshape(-1) > 0.0, cn.reshape(-1), 1e30)
    fixg_t = fixgweights.reshape(-1)
    refqs_t = (refq * spq).reshape(-1)
    gamc_t = gam * G_C
    c6_pad = jnp.pad(refc6, ((0, 0), (0, 0), (0, 1), (0, 1)))
    c6_bf = c6_pad.reshape(NPAIR, 32, 2).astype(jnp.bfloat16)
    c6_t = lax.bitcast_convert_type(c6_bf, i32)

    zp = jnp.concatenate([Z.astype(i32),
                          1 + (jnp.arange(NA_PAD - NA, dtype=i32) % (ZMAX - 1))])
    qap = jnp.concatenate([qa, jnp.zeros((NA_PAD - NA,), f32)])
    pad_ii = NA + (jnp.arange(E_PAD - NE, dtype=i32) % (NA_PAD - NA))
    iip = jnp.concatenate([idx_i.astype(i32), pad_ii])
    pad_jj = jnp.arange(E_PAD - NE, dtype=i32) % NA
    jjp = jnp.concatenate([idx_j.astype(i32), pad_jj])
    rp = jnp.concatenate([r_ij, jnp.full((E_PAD - NE,), 1e9, f32)])
    iip2 = iip.reshape(EROWS, 128)
    jjp2 = jjp.reshape(EROWS, 128)
    rp2 = rp.reshape(EROWS, 128)
    los = jnp.minimum(iip[:: EDGES_W], NA_PAD - RMAX)

    phase_a, phase_b, phase_c = _build_phases()
    cov2, pp2 = phase_a(zp, iip2, jjp2, rp2, rco_t, den_t, los)
    zeta = phase_b(cov2, zp, qap, wt_t, cn_t, fixg_t, refqs_t,
                   zeff, gamc_t, kvec)
    ed2 = phase_c(iip2, jjp2, rp2, pp2, zeta, c6_t, p6_t, p8_t, w8_t, los)
    edisp = ed2[:NA] + ed2[NA_PAD:NA_PAD + NA]
    zeros = jnp.zeros((NA,), f32)
    return edisp, zeros, zeros


# speedup vs baseline: 1.4088x; 1.4088x over previous
import jax
import jax.numpy as jnp
from jax import lax
from jax.experimental import pallas as pl
from jax.experimental.pallas import tpu as pltpu
from jax.experimental.pallas import tpu_sc as plsc

ZMAX = 87
NREF = 7
G_A = 3.0
G_C = 2.0
K2 = 4.0 / 3.0
K4 = 4.10451
K5 = 19.08857
K6 = 254.5553148552
KN = 7.5
WF = 6.0
BOHR = 0.5291772105638411
HARTREE = 27.211386024367243
C2B = 1.0 / BOHR
C2EV = 0.5 * HARTREE
EGA = 20.085536923187668

NA = 50000
NE = 800000
NW = 32
ATOMS_W = 1568
NA_PAD = NW * ATOMS_W
ZSL = NA_PAD // 16
CH = 512
RPC = CH // 128
EDGES_W = 25600
E_PAD = NW * EDGES_W
EROWS = E_PAD // 128
NC = EDGES_W // CH
NPAIR = ZMAX * ZMAX
N49 = ZMAX * 49
C6W = 56
RMAX = 4096


def _erf(x):
    ax = jnp.abs(x)
    t = 1.0 / (1.0 + 0.3275911 * ax)
    poly = t * (0.254829592 + t * (-0.284496736 + t * (1.421413741
               + t * (-1.453152027 + t * 1.061405429))))
    e = 1.0 - poly * jnp.exp(-x * x)
    return jnp.where(x >= 0, e, -e)


def _zero_f32(ref, n):
    def z16(i, carry):
        ref[pl.ds(i * 16, 16)] = jnp.zeros((16,), jnp.float32)
        return carry
    lax.fori_loop(0, n // 16, z16, 0)


def _lo_vec(los_v, w):
    return plsc.load_gather(los_v, [jnp.full((16,), w, jnp.int32)])


def _merge_acc(acc_v, ib_v, sh, lo, sem):
    ar16 = lax.iota(jnp.int32, 16)
    for b in range(RMAX // 128):
        for k in range(8):
            ib_v[b, pl.ds(k * 16, 16)] = lo + (b * 128 + k * 16) + ar16
    cps = [pltpu.async_copy(acc_v.at[pl.ds(b * 128, 128)],
                            sh.at[ib_v.at[b]], sem, add=True)
           for b in range(RMAX // 128)]
    for c in cps:
        c.wait()


def _readback(sl_v, sh, out, sid, cid):
    pltpu.sync_copy(sh.at[pl.ds(sid * ZSL, ZSL)], sl_v)
    pltpu.sync_copy(sl_v, out.at[pl.ds(cid * NA_PAD + sid * ZSL, ZSL)])


def _phase_a_body(zp, iip2, jjp2, rp2, rcot, dent, los, outc, outp,
                  z_v, rco_v, den_v, los_v,
                  iiA, jjA, rA, pbA, iiB, jjB, rB, pbB,
                  acc_v, ib_v, sl_v, cov_sh, semA, semB, semM):
    cid = lax.axis_index("c")
    sid = lax.axis_index("s")
    w = cid * 16 + sid
    _zero_f32(sl_v, ZSL)
    pltpu.sync_copy(sl_v, cov_sh.at[pl.ds(sid * ZSL, ZSL)])
    _zero_f32(acc_v, RMAX)
    pltpu.sync_copy(zp, z_v)
    pltpu.sync_copy(rcot, rco_v)
    pltpu.sync_copy(dent, den_v)
    pltpu.sync_copy(los, los_v)
    lo = _lo_vec(los_v, w)
    plsc.subcore_barrier()

    def rows0(c):
        return w * (EDGES_W // 128) + jnp.minimum(c, NC - 1) * RPC

    def load(c, ii2, jj2, r2, sem, fire):
        r0 = rows0(c)
        args = [(iip2, ii2), (jjp2, jj2), (rp2, r2)]
        for src, dst in args:
            if fire:
                pltpu.async_copy(src.at[pl.ds(r0, RPC), :], dst, sem)
            else:
                pltpu.make_async_copy(src.at[pl.ds(r0, RPC), :], dst, sem).wait()

    def compute(c, ii2, jj2, r2, pb):
        def vreg(v, carry):
            j = v // 8
            o = (v % 8) * 16
            ii = ii2[j, pl.ds(o, 16)]
            jj = jj2[j, pl.ds(o, 16)]
            zi = plsc.load_gather(z_v, [ii])
            zj = plsc.load_gather(z_v, [jj])
            p = zi * ZMAX + zj
            pb[j, pl.ds(o, 16)] = p
            rcoi = plsc.load_gather(rco_v, [p])
            den = plsc.load_gather(den_v, [p])
            r = r2[j, pl.ds(o, 16)] * C2B
            tmp = den * 0.5 * (1.0 + _erf(KN * (1.0 - r * rcoi)))
            plsc.addupdate_scatter(acc_v, [ii - lo], tmp)
            return carry
        lax.fori_loop(0, CH // 16, vreg, 0)

    def pwrite(c, pb, sem, fire):
        dst = outp.at[pl.ds(rows0(c), RPC), :]
        if fire:
            pltpu.async_copy(pb, dst, sem)
        else:
            pltpu.make_async_copy(pb, dst, sem).wait()

    load(0, iiA, jjA, rA, semA, True)

    def pair(t, carry):
        c0 = 2 * t
        load(c0, iiA, jjA, rA, semA, False)
        load(c0 + 1, iiB, jjB, rB, semB, True)
        compute(c0, iiA, jjA, rA, pbA)
        pwrite(c0, pbA, semM, True)
        load(c0 + 1, iiB, jjB, rB, semB, False)
        load(c0 + 2, iiA, jjA, rA, semA, True)
        compute(c0 + 1, iiB, jjB, rB, pbB)
        pwrite(c0 + 1, pbB, semM, True)
        pwrite(c0, pbA, semM, False)
        pwrite(c0 + 1, pbB, semM, False)
        return carry

    lax.fori_loop(0, NC // 2, pair, 0)
    load(NC - 1, iiA, jjA, rA, semA, False)
    _merge_acc(acc_v, ib_v, cov_sh, lo, semM)
    plsc.subcore_barrier()
    _readback(sl_v, cov_sh, outc, sid, cid)


def _phase_b_body(cov2, zp, qap, wtt, cnt, fixgt, refqst, zefft, gamt,
                  kvec, out, covA, covB, zc, qac, wt_v, cn_v, fixg_v,
                  refqs_v, zeff_v, gam_v, kv_v, zbuf):
    w = lax.axis_index("c") * 16 + lax.axis_index("s")
    a0 = w * ATOMS_W
    pltpu.sync_copy(cov2.at[pl.ds(a0, ATOMS_W)], covA)
    pltpu.sync_copy(cov2.at[pl.ds(NA_PAD + a0, ATOMS_W)], covB)
    pltpu.sync_copy(zp.at[pl.ds(a0, ATOMS_W)], zc)
    pltpu.sync_copy(qap.at[pl.ds(a0, ATOMS_W)], qac)
    pltpu.sync_copy(wtt, wt_v)
    pltpu.sync_copy(cnt, cn_v)
    pltpu.sync_copy(fixgt, fixg_v)
    pltpu.sync_copy(refqst, refqs_v)
    pltpu.sync_copy(zefft, zeff_v)
    pltpu.sync_copy(gamt, gam_v)
    pltpu.sync_copy(kvec, kv_v)
    ar16 = lax.iota(jnp.int32, 16)

    def atoms(t, carry):
        o = t * 16
        cov = covA[pl.ds(o, 16)] + covB[pl.ds(o, 16)]
        zv = zc[pl.ds(o, 16)]
        b49 = zv * 49
        b7 = zv * NREF
        iz = plsc.load_gather(zeff_v, [zv])
        qmod = iz + qac[pl.ds(o, 16)]
        ok = qmod > 1e-8
        qm = jnp.where(ok, qmod, 1.0)
        g = plsc.load_gather(gam_v, [zv])
        gws = []
        norm = jnp.zeros((16,), jnp.float32)
        for i in range(NREF):
            acc = jnp.zeros((16,), jnp.float32)
            for j in range(NREF):
                idx = b49 + (NREF * i + j)
                wv = plsc.load_gather(wt_v, [idx])
                cnv = plsc.load_gather(cn_v, [idx])
                d = cov - cnv
                acc = acc + jnp.exp(-wv * d * d)
            gws.append(acc)
            norm = norm + acc
        nok = norm > 1e-8
        nrm = jnp.where(nok, norm, 1.0)
        kv = kv_v[pl.ds(0, 16)]
        for i in range(NREF):
            qref = iz + plsc.load_gather(refqs_v, [b7 + i])
            zt = jnp.where(ok, jnp.exp(G_A * (1.0 - jnp.exp(g * (1.0 - qref / qm)))), EGA)
            gwi = jnp.where(nok, gws[i] / nrm, plsc.load_gather(fixg_v, [b7 + i]))
            z = zt * gwi * kv
            plsc.store_scatter(zbuf, [o + ar16, jnp.full((16,), i, jnp.int32)], z)
        return carry

    lax.fori_loop(0, ATOMS_W // 16, atoms, 0)
    pltpu.sync_copy(zbuf, out.at[pl.ds(a0, ATOMS_W), :])


def _phase_c_body(iip2, jjp2, rp2, pp2, zeta, c6t, p6t, p8t, w8t, los, out,
                  p6_v, p8_v, w8_v, los_v,
                  iiA, jjA, ppA, rA, iiB, jjB, ppB, rB,
                  ziA, zjA, c6A, ziB, zjB, c6B,
                  acc_v, ib_v, sl_v, ed_sh,
                  semLA, semLB, semGA, semGB, semM):
    cid = lax.axis_index("c")
    sid = lax.axis_index("s")
    w = cid * 16 + sid
    _zero_f32(sl_v, ZSL)
    pltpu.sync_copy(sl_v, ed_sh.at[pl.ds(sid * ZSL, ZSL)])
    _zero_f32(acc_v, RMAX)
    pltpu.sync_copy(p6t, p6_v)
    pltpu.sync_copy(p8t, p8_v)
    pltpu.sync_copy(w8t, w8_v)
    pltpu.sync_copy(los, los_v)
    lo = _lo_vec(los_v, w)
    plsc.subcore_barrier()
    ar16 = lax.iota(jnp.int32, 16)

    def rows0(c):
        return w * (EDGES_W // 128) + jnp.minimum(c, NC - 1) * RPC

    def load(c, ii2, jj2, pp2v, r2, sem, fire):
        r0 = rows0(c)
        for src, dst in [(iip2, ii2), (jjp2, jj2), (pp2, pp2v), (rp2, r2)]:
            if fire:
                pltpu.async_copy(src.at[pl.ds(r0, RPC), :], dst, sem)
            else:
                pltpu.make_async_copy(src.at[pl.ds(r0, RPC), :], dst, sem).wait()

    def gath(ii2, jj2, pp2v, zi_v, zj_v, c6_v, sem, fire):
        for j in range(RPC):
            trips = [(zeta, ii2, zi_v, 8), (zeta, jj2, zj_v, 8),
                     (c6t, pp2v, c6_v, C6W)]
            for tab, idx2, dst, _d in trips:
                src = tab.at[idx2.at[j]]
                dsl = dst.at[pl.ds(j * 128, 128), :]
                if fire:
                    pltpu.async_copy(src, dsl, sem)
                else:
                    pltpu.make_async_copy(src, dsl, sem).wait()

    def compute(ii2, pp2v, r2, zi_v, zj_v, c6_v):
        def one(v):
            j = v // 8
            o = (v % 8) * 16
            rows = v * 16 + ar16
            ii = ii2[j, pl.ds(o, 16)]
            p = pp2v[j, pl.ds(o, 16)]
            r = r2[j, pl.ds(o, 16)] * C2B
            r2_ = r * r
            r4 = r2_ * r2_
            r6 = r4 * r2_
            r8 = r4 * r4
            p6 = plsc.load_gather(p6_v, [p])
            p8 = plsc.load_gather(p8_v, [p])
            w8 = plsc.load_gather(w8_v, [p])
            oor6 = 1.0 / (r6 + p6)
            oor8 = 1.0 / (r8 + p8)
            zjb = [plsc.load_gather(zj_v, [rows, jnp.full((16,), b, jnp.int32)])
                   for b in range(NREF)]
            c6 = jnp.zeros((16,), jnp.float32)
            for a in range(NREF):
                zia = plsc.load_gather(zi_v, [rows, jnp.full((16,), a, jnp.int32)])
                s = jnp.zeros((16,), jnp.float32)
                for b in range(NREF):
                    cr = plsc.load_gather(
                        c6_v, [rows, jnp.full((16,), a * NREF + b, jnp.int32)])
                    s = s + cr * zjb[b]
                c6 = c6 + zia * s
            pw = -c6 * (oor6 + w8 * oor8)
            plsc.addupdate_scatter(acc_v, [ii - lo], pw)

        def vreg2(u, carry):
            one(2 * u)
            one(2 * u + 1)
            return carry
        lax.fori_loop(0, CH // 32, vreg2, 0)

    LA = (iiA, jjA, ppA, rA)
    LB = (iiB, jjB, ppB, rB)
    GA = (ziA, zjA, c6A)
    GB = (ziB, zjB, c6B)

    load(0, *LA, semLA, True)
    load(0, *LA, semLA, False)
    gath(iiA, jjA, ppA, *GA, semGA, True)
    load(1, *LB, semLB, True)

    def pair(t, carry):
        c0 = 2 * t
        load(c0 + 1, *LB, semLB, False)
        gath(iiB, jjB, ppB, *GB, semGB, True)
        gath(iiA, jjA, ppA, *GA, semGA, False)
        compute(iiA, ppA, rA, *GA)
        load(c0 + 2, *LA, semLA, True)
        load(c0 + 2, *LA, semLA, False)
        gath(iiA, jjA, ppA, *GA, semGA, True)
        gath(iiB, jjB, ppB, *GB, semGB, False)
        compute(iiB, ppB, rB, *GB)
        load(c0 + 3, *LB, semLB, True)
        return carry

    lax.fori_loop(0, NC // 2, pair, 0)
    load(NC - 1, *LB, semLB, False)
    gath(iiA, jjA, ppA, *GA, semGA, False)
    _merge_acc(acc_v, ib_v, ed_sh, lo, semM)
    plsc.subcore_barrier()
    _readback(sl_v, ed_sh, out, sid, cid)


def _build_phases():
    mesh = plsc.VectorSubcoreMesh(core_axis_name="c", subcore_axis_name="s")
    f32 = jnp.float32
    i32 = jnp.int32
    cparams = pltpu.CompilerParams(needs_layout_passes=False,
                                   use_tc_tiling_on_sc=False)
    phase_a = pl.kernel(
        _phase_a_body,
        out_type=(jax.ShapeDtypeStruct((2 * NA_PAD,), f32),
                  jax.ShapeDtypeStruct((EROWS, 128), i32)),
        compiler_params=cparams,
        mesh=mesh,
        scratch_types=[
            pltpu.VMEM((NA_PAD,), i32),
            pltpu.VMEM((NPAIR,), f32),
            pltpu.VMEM((NPAIR,), f32),
            pltpu.VMEM((NW,), i32),
            pltpu.VMEM((RPC, 128), i32),
            pltpu.VMEM((RPC, 128), i32),
            pltpu.VMEM((RPC, 128), f32),
            pltpu.VMEM((RPC, 128), i32),
            pltpu.VMEM((RPC, 128), i32),
            pltpu.VMEM((RPC, 128), i32),
            pltpu.VMEM((RPC, 128), f32),
            pltpu.VMEM((RPC, 128), i32),
            pltpu.VMEM((RMAX,), f32),
            pltpu.VMEM((RMAX // 128, 128), i32),
            pltpu.VMEM((ZSL,), f32),
            pltpu.VMEM_SHARED((NA_PAD,), f32),
            pltpu.SemaphoreType.DMA,
            pltpu.SemaphoreType.DMA,
            pltpu.SemaphoreType.DMA,
        ],
    )
    phase_b = pl.kernel(
        _phase_b_body,
        out_type=jax.ShapeDtypeStruct((NA_PAD, 8), f32),
        compiler_params=cparams,
        mesh=mesh,
        scratch_types=[
            pltpu.VMEM((ATOMS_W,), f32),
            pltpu.VMEM((ATOMS_W,), f32),
            pltpu.VMEM((ATOMS_W,), i32),
            pltpu.VMEM((ATOMS_W,), f32),
            pltpu.VMEM((N49,), f32),
            pltpu.VMEM((N49,), f32),
            pltpu.VMEM((ZMAX * NREF,), f32),
            pltpu.VMEM((ZMAX * NREF,), f32),
            pltpu.VMEM((ZMAX,), f32),
            pltpu.VMEM((ZMAX,), f32),
            pltpu.VMEM((16,), f32),
            pltpu.VMEM((ATOMS_W, 8), f32),
        ],
    )
    phase_c = pl.kernel(
        _phase_c_body,
        out_type=jax.ShapeDtypeStruct((2 * NA_PAD,), f32),
        compiler_params=cparams,
        mesh=mesh,
        scratch_types=[
            pltpu.VMEM((NPAIR,), f32),
            pltpu.VMEM((NPAIR,), f32),
            pltpu.VMEM((NPAIR,), f32),
            pltpu.VMEM((NW,), i32),
            pltpu.VMEM((RPC, 128), i32),
            pltpu.VMEM((RPC, 128), i32),
            pltpu.VMEM((RPC, 128), i32),
            pltpu.VMEM((RPC, 128), f32),
            pltpu.VMEM((RPC, 128), i32),
            pltpu.VMEM((RPC, 128), i32),
            pltpu.VMEM((RPC, 128), i32),
            pltpu.VMEM((RPC, 128), f32),
            pltpu.VMEM((CH, 8), f32),
            pltpu.VMEM((CH, 8), f32),
            pltpu.VMEM((CH, C6W), f32),
            pltpu.VMEM((CH, 8), f32),
            pltpu.VMEM((CH, 8), f32),
            pltpu.VMEM((CH, C6W), f32),
            pltpu.VMEM((RMAX,), f32),
            pltpu.VMEM((RMAX // 128, 128), i32),
            pltpu.VMEM((ZSL,), f32),
            pltpu.VMEM_SHARED((NA_PAD,), f32),
            pltpu.SemaphoreType.DMA,
            pltpu.SemaphoreType.DMA,
            pltpu.SemaphoreType.DMA,
            pltpu.SemaphoreType.DMA,
            pltpu.SemaphoreType.DMA,
        ],
    )
    return phase_a, phase_b, phase_c


def kernel(Z, idx_i, idx_j, r_ij, qa, s6_raw, s8_raw, a1_raw, a2_raw,
           scaleq_raw, refc6, rcov, en, ncount_mask, ncount_weight, cn,
           fixgweights, refq, zeff, gam, sqrt_r4r2):
    f32 = jnp.float32
    i32 = jnp.int32
    s6 = jax.nn.softplus(s6_raw)
    s8 = jax.nn.softplus(s8_raw)
    a1 = jax.nn.softplus(a1_raw)
    a2 = jax.nn.softplus(a2_raw)
    spq = jax.nn.softplus(scaleq_raw)

    rco_t = (1.0 / (K2 * (rcov[:, None] + rcov[None, :]))).reshape(-1)
    den_t = (K4 * jnp.exp(-(jnp.abs(en[:, None] - en[None, :]) + K5) ** 2 / K6)).reshape(-1)
    r4_t = (jnp.float32(3.0 ** 0.5) * sqrt_r4r2[:, None] * sqrt_r4r2[None, :]).reshape(-1)
    r0_t = a1 * r4_t + a2
    p6_t = r0_t ** 6
    p8_t = r0_t ** 8
    w8_t = s8 * r4_t * r4_t / s6
    kvec = jnp.full((16,), jnp.sqrt(s6 * C2EV), f32)

    wt_t = (WF * ncount_weight).reshape(-1)
    cn_t = jnp.where(ncount_mask.reshape(-1) > 0.0, cn.reshape(-1), 1e30)
    fixg_t = fixgweights.reshape(-1)
    refqs_t = (refq * spq).reshape(-1)
    gamc_t = gam * G_C
    c6_t = jnp.pad(refc6.reshape(NPAIR, 49), ((0, 0), (0, C6W - 49)))

    zp = jnp.concatenate([Z.astype(i32),
                          1 + (jnp.arange(NA_PAD - NA, dtype=i32) % (ZMAX - 1))])
    qap = jnp.concatenate([qa, jnp.zeros((NA_PAD - NA,), f32)])
    pad_ii = NA + (jnp.arange(E_PAD - NE, dtype=i32) % (NA_PAD - NA))
    iip = jnp.concatenate([idx_i.astype(i32), pad_ii])
    pad_jj = jnp.arange(E_PAD - NE, dtype=i32) % NA
    jjp = jnp.concatenate([idx_j.astype(i32), pad_jj])
    rp = jnp.concatenate([r_ij, jnp.full((E_PAD - NE,), 1e9, f32)])
    iip2 = iip.reshape(EROWS, 128)
    jjp2 = jjp.reshape(EROWS, 128)
    rp2 = rp.reshape(EROWS, 128)
    los = jnp.minimum(iip[:: EDGES_W], NA_PAD - RMAX)

    phase_a, phase_b, phase_c = _build_phases()
    cov2, pp2 = phase_a(zp, iip2, jjp2, rp2, rco_t, den_t, los)
    zeta = phase_b(cov2, zp, qap, wt_t, cn_t, fixg_t, refqs_t,
                   zeff, gamc_t, kvec)
    ed2 = phase_c(iip2, jjp2, rp2, pp2, zeta, c6_t, p6_t, p8_t, w8_t, los)
    edisp = ed2[:NA] + ed2[NA_PAD:NA_PAD + NA]
    zeros = jnp.zeros((NA,), f32)
    return edisp, zeros, zeros
